# Initial kernel scaffold; baseline (speedup 1.0000x reference)
import jax
import jax.numpy as jnp
from jax.experimental import pallas as pl


def kernel(x, edge_index, W1a, b1a, W1b, b1b, W2a, b2a, W2b, b2b):
    raise NotImplementedError("write your pallas kernel here")


# speedup vs baseline: 32.4792x; 32.4792x over previous
import functools

import jax
import jax.numpy as jnp
from jax import lax
from jax.experimental import pallas as pl
from jax.experimental.pallas import tpu as pltpu
from jax.experimental.pallas import tpu_sc as plsc

N = 100000
E = 6400000
F = 16
N_PAD = 102400
B = 80
K = 10
NW = 32
E_PER_W = E // NW
OUTER = E_PER_W // (K * B)
ZROWS = 640
RPT = N_PAD // 16


def _make_agg_kernel():
    mesh = plsc.VectorSubcoreMesh(core_axis_name="c", subcore_axis_name="s")

    @functools.partial(
        pl.kernel,
        out_type=jax.ShapeDtypeStruct((2, N_PAD, F), jnp.float32),
        scratch_types=[
            pltpu.VMEM((K * B,), jnp.int32),
            pltpu.VMEM((K * B,), jnp.int32),
            pltpu.VMEM((K * B, F), jnp.float32),
            pltpu.VMEM((ZROWS, F), jnp.float32),
            pltpu.VMEM_SHARED((N_PAD, F), jnp.float32),
            pltpu.SemaphoreType.DMA,
        ],
        mesh=mesh,
        compiler_params=pltpu.CompilerParams(use_tc_tiling_on_sc=False),
    )
    def agg(src_hbm, dst_hbm, table_hbm, out_hbm, sidx, didx, rows, zbuf, acc, sem):
        c = lax.axis_index("c")
        s = lax.axis_index("s")
        w = s * 2 + c

        def zero_row(i, _):
            zbuf[i, :] = jnp.zeros((F,), jnp.float32)
            return 0
        lax.fori_loop(0, ZROWS, zero_row, 0)

        def zero_copy(i, _):
            pltpu.sync_copy(zbuf, acc.at[pl.ds(s * RPT + i * ZROWS, ZROWS)])
            return 0
        lax.fori_loop(0, RPT // ZROWS, zero_copy, 0)

        plsc.subcore_barrier()

        def step(it, _):
            base = w * E_PER_W + it * (K * B)
            pltpu.sync_copy(src_hbm.at[pl.ds(base, K * B)], sidx)
            pltpu.sync_copy(dst_hbm.at[pl.ds(base, K * B)], didx)
            cps = [
                pltpu.async_copy(
                    table_hbm.at[sidx.at[pl.ds(j * B, B)]],
                    rows.at[pl.ds(j * B, B)],
                    sem,
                )
                for j in range(K)
            ]
            for cp in cps:
                cp.wait()
            for j in range(K):
                pltpu.sync_copy(
                    rows.at[pl.ds(j * B, B)],
                    acc.at[didx.at[pl.ds(j * B, B)]],
                    add=True,
                )
            return 0
        lax.fori_loop(0, OUTER, step, 0)

        plsc.subcore_barrier()

        pltpu.sync_copy(
            acc.at[pl.ds(s * RPT, RPT)],
            out_hbm.at[c, pl.ds(s * RPT, RPT)],
        )

    return agg


_agg = _make_agg_kernel()


M_PK = N_PAD // 8


def _mlp_body(h_ref, p0_ref, p1_ref, wa_ref, ba_ref, wb_ref, bb_ref, o_ref):
    g = h_ref[...] + p0_ref[...] + p1_ref[...]
    t = jnp.dot(g, wa_ref[...], preferred_element_type=jnp.float32) + ba_ref[...]
    t = jnp.maximum(t, 0.0)
    o_ref[...] = jnp.dot(t, wb_ref[...], preferred_element_type=jnp.float32) + bb_ref[...]


def _mlp(h, p0, p1, wa_blk, ba_t, wb_blk, bb_t):
    bm = 1600
    grid = (M_PK // bm,)
    row_spec = pl.BlockSpec((bm, 128), lambda i: (i, 0))
    w_spec = pl.BlockSpec((128, 128), lambda i: (0, 0))
    b_spec = pl.BlockSpec((1, 128), lambda i: (0, 0))
    return pl.pallas_call(
        _mlp_body,
        grid=grid,
        in_specs=[row_spec, row_spec, row_spec, w_spec, b_spec, w_spec, b_spec],
        out_specs=row_spec,
        out_shape=jax.ShapeDtypeStruct((M_PK, 128), jnp.float32),
    )(h, p0, p1, wa_blk, ba_t, wb_blk, bb_t)


def _blk(w):
    return jnp.kron(jnp.eye(8, dtype=jnp.float32), w)


def _tile(b):
    return jnp.tile(b.reshape(1, F), (1, 8))


def kernel(x, edge_index, W1a, b1a, W1b, b1b, W2a, b2a, W2b, b2b):
    f_in = x.shape[1]
    xp = jnp.pad(x, ((0, N_PAD - N), (0, F - f_in)))
    W1ap = jnp.pad(W1a, ((0, F - f_in), (0, 0)))
    ei = edge_index.astype(jnp.int32)
    src1 = ei[0]
    dst1 = ei[1]
    xpk = xp.reshape(M_PK, 128)

    parts1 = _agg(src1, dst1, xp)
    p1k = parts1.reshape(2, M_PK, 128)
    h1k = _mlp(xpk, p1k[0], p1k[1], _blk(W1ap), _tile(b1a), _blk(W1b), _tile(b1b))
    h1 = h1k.reshape(N_PAD, F)
    parts2 = _agg(src1, dst1, h1)
    p2k = parts2.reshape(2, M_PK, 128)
    outk = _mlp(h1k, p2k[0], p2k[1], _blk(W2a), _tile(b2a), _blk(W2b), _tile(b2b))
    return outk.reshape(N_PAD, F)[:N]
